# baseline (speedup 1.0000x reference)
import jax
import jax.numpy as jnp
from jax.experimental import pallas as pl
from jax.experimental.pallas import tpu as pltpu

E = 64
K = 8


def _router_block(x_ref, w_ref, b_ref, scores_ref, idx_ref):
    x = x_ref[...]
    w = w_ref[...]
    lt = jax.lax.dot_general(
        w, x, (((1,), (1,)), ((), ())),
        preferred_element_type=jnp.float32)
    lt = lt + b_ref[...][:, None]

    m = jnp.max(lt, axis=0, keepdims=True)
    e = jnp.exp(lt - m)
    scores_t = e / jnp.sum(e, axis=0, keepdims=True)
    scores_ref[...] = scores_t.T

    tb = scores_t.shape[1]
    iota = jax.lax.broadcasted_iota(jnp.int32, (E, tb), 0)
    s = scores_t
    neg = jnp.float32(-jnp.inf)
    for k in range(K):
        mk = jnp.max(s, axis=0, keepdims=True)
        cand = jnp.where(s == mk, iota, E)
        amin = jnp.min(cand, axis=0, keepdims=True)
        idx_ref[k, :] = amin[0]
        s = jnp.where(iota == amin, neg, s)


def kernel(hidden_states, weight, bias):
    Bn, Sn, Hn = hidden_states.shape
    T = Bn * Sn
    flat = hidden_states.reshape(T, Hn)
    TB = 512
    grid = (T // TB,)

    scores, idx_t = pl.pallas_call(
        _router_block,
        grid=grid,
        in_specs=[
            pl.BlockSpec((TB, Hn), lambda i: (i, 0)),
            pl.BlockSpec((E, Hn), lambda i: (0, 0)),
            pl.BlockSpec((E,), lambda i: (0,)),
        ],
        out_specs=[
            pl.BlockSpec((TB, E), lambda i: (i, 0)),
            pl.BlockSpec((K, TB), lambda i: (0, i)),
        ],
        out_shape=[
            jax.ShapeDtypeStruct((T, E), jnp.float32),
            jax.ShapeDtypeStruct((K, T), jnp.int32),
        ],
        compiler_params=pltpu.CompilerParams(
            dimension_semantics=("arbitrary",),
        ),
    )(flat, weight, bias)
    return (scores, idx_t.T)


# speedup vs baseline: 1.0523x; 1.0523x over previous
import jax
import jax.numpy as jnp
from jax.experimental import pallas as pl
from jax.experimental.pallas import tpu as pltpu

E = 64
K = 8


def _router_block(x1_ref, x2_ref, w_ref, b_ref, scores_ref, idx_ref):
    x1 = x1_ref[...]
    x2 = x2_ref[...]
    w = w_ref[...]
    hh = x1.shape[1]
    lt = jax.lax.dot_general(
        w[:, :hh], x1, (((1,), (1,)), ((), ())),
        preferred_element_type=jnp.float32)
    lt = lt + jax.lax.dot_general(
        w[:, hh:], x2, (((1,), (1,)), ((), ())),
        preferred_element_type=jnp.float32)
    lt = lt + b_ref[...][:, None]

    m = jnp.max(lt, axis=0, keepdims=True)
    e = jnp.exp(lt - m)
    scores_t = e / jnp.sum(e, axis=0, keepdims=True)
    scores_ref[...] = scores_t.T

    tb = scores_t.shape[1]
    iota = jax.lax.broadcasted_iota(jnp.int32, (E, tb), 0)
    s = scores_t
    neg = jnp.float32(-jnp.inf)
    for k in range(K):
        mk = jnp.max(s, axis=0, keepdims=True)
        cand = jnp.where(s == mk, iota, E)
        amin = jnp.min(cand, axis=0, keepdims=True)
        idx_ref[k, :] = amin[0]
        s = jnp.where(iota == amin, neg, s)


def kernel(hidden_states, weight, bias):
    Bn, Sn, Hn = hidden_states.shape
    T = Bn * Sn
    hh = Hn // 2
    flat = hidden_states.reshape(T, Hn)
    TB = 1024
    grid = (T // TB,)

    scores, idx_t = pl.pallas_call(
        _router_block,
        grid=grid,
        in_specs=[
            pl.BlockSpec((TB, hh), lambda i: (i, 0)),
            pl.BlockSpec((TB, hh), lambda i: (i, 1)),
            pl.BlockSpec((E, Hn), lambda i: (0, 0)),
            pl.BlockSpec((E,), lambda i: (0,)),
        ],
        out_specs=[
            pl.BlockSpec((TB, E), lambda i: (i, 0)),
            pl.BlockSpec((K, TB), lambda i: (0, i)),
        ],
        out_shape=[
            jax.ShapeDtypeStruct((T, E), jnp.float32),
            jax.ShapeDtypeStruct((K, T), jnp.int32),
        ],
        compiler_params=pltpu.CompilerParams(
            dimension_semantics=("arbitrary",),
        ),
    )(flat, flat, weight, bias)
    return (scores, idx_t.T)
